# baseline (speedup 1.0000x reference)
import functools

import jax
import jax.numpy as jnp
from jax import lax
from jax.experimental import pallas as pl
from jax.experimental.pallas import tpu as pltpu
from jax.experimental.pallas import tpu_sc as plsc

B = 4096 * 200
D = 128
NC, NS = 2, 16
NW = NC * NS
BPW = B // NW
C = 320
NCHUNK = BPW // C
NBUF = 2


def _sc_gather(idx_flat, ne):
    mesh = plsc.VectorSubcoreMesh(core_axis_name="c", subcore_axis_name="s")

    @functools.partial(
        pl.kernel,
        mesh=mesh,
        out_type=jax.ShapeDtypeStruct((B, D), jnp.float32),
        scratch_types=(
            [pltpu.VMEM((BPW,), jnp.int32)]
            + [pltpu.VMEM((C, D), jnp.float32) for _ in range(NBUF)]
            + [pltpu.VMEM_SHARED((512, D), jnp.float32)]
            + [pltpu.SemaphoreType.DMA for _ in range(2 * NBUF)]
        ),
    )
    def k(idx_hbm, table_hbm, out_hbm, idx_all, *rest):
        rows = rest[:NBUF]
        table_spm = rest[NBUF]
        gsem = rest[NBUF + 1:NBUF + 1 + NBUF]
        ssem = rest[NBUF + 1 + NBUF:]

        wid = lax.axis_index("s") * NC + lax.axis_index("c")
        base = wid * BPW

        @pl.when(lax.axis_index("s") == 0)
        def _():
            pltpu.sync_copy(table_hbm, table_spm)

        pltpu.sync_copy(idx_hbm.at[pl.ds(base, BPW)], idx_all)
        plsc.subcore_barrier()

        def gather_desc(i, b):
            return pltpu.make_async_copy(
                table_spm.at[idx_all.at[pl.ds(i * C, C)]], rows[b], gsem[b])

        def store_desc(i, b):
            return pltpu.make_async_copy(
                rows[b], out_hbm.at[pl.ds(base + i * C, C)], ssem[b])

        for b in range(NBUF - 1):
            gather_desc(b, b).start()

        for b in range(NBUF):
            gather_desc(b, b).wait()
            store_desc(b, b).start()
            if b > 0:
                store_desc(b - 1, b - 1).wait()
            gather_desc(b + NBUF - 1, (b - 1) % NBUF).start()

        def body(g, carry):
            for b in range(NBUF):
                i = NBUF * g + b
                gather_desc(i, b).wait()
                store_desc(i, b).start()
                store_desc(i - 1, (b - 1) % NBUF).wait()
                gather_desc(i + NBUF - 1, (b - 1) % NBUF).start()
            return carry

        lax.fori_loop(1, NCHUNK // NBUF - 1, body, 0)

        for b in range(NBUF):
            i = NCHUNK - NBUF + b
            gather_desc(i, b).wait()
            store_desc(i, b).start()
            store_desc(i - 1, (b - 1) % NBUF).wait()
            if b == 0:
                gather_desc(i + NBUF - 1, (b - 1) % NBUF).start()
        store_desc(NCHUNK - 1, NBUF - 1).wait()

    return k(idx_flat, ne)


def kernel(node_ids, ne):
    idx_flat = node_ids.reshape(-1).astype(jnp.int32)
    out = _sc_gather(idx_flat, ne)
    return out.reshape(node_ids.shape + (D,))


# speedup vs baseline: 1.0322x; 1.0322x over previous
import functools

import jax
import jax.numpy as jnp
from jax import lax
from jax.experimental import pallas as pl
from jax.experimental.pallas import tpu as pltpu
from jax.experimental.pallas import tpu_sc as plsc

B = 4096 * 200
D = 128
NC, NS = 2, 16
NW = NC * NS
BPW = B // NW
C = 160
NCHUNK = BPW // C
NBUF = 4


def _sc_gather(idx_flat, ne):
    mesh = plsc.VectorSubcoreMesh(core_axis_name="c", subcore_axis_name="s")

    @functools.partial(
        pl.kernel,
        mesh=mesh,
        out_type=jax.ShapeDtypeStruct((B, D), jnp.float32),
        scratch_types=(
            [pltpu.VMEM((BPW,), jnp.int32)]
            + [pltpu.VMEM((C, D), jnp.float32) for _ in range(NBUF)]
            + [pltpu.VMEM_SHARED((512, D), jnp.float32)]
            + [pltpu.SemaphoreType.DMA for _ in range(2 * NBUF)]
        ),
    )
    def k(idx_hbm, table_hbm, out_hbm, idx_all, *rest):
        rows = rest[:NBUF]
        table_spm = rest[NBUF]
        gsem = rest[NBUF + 1:NBUF + 1 + NBUF]
        ssem = rest[NBUF + 1 + NBUF:]

        wid = lax.axis_index("s") * NC + lax.axis_index("c")
        base = wid * BPW

        @pl.when(lax.axis_index("s") == 0)
        def _():
            pltpu.sync_copy(table_hbm, table_spm)

        pltpu.sync_copy(idx_hbm.at[pl.ds(base, BPW)], idx_all)
        plsc.subcore_barrier()

        def gather_desc(i, b):
            return pltpu.make_async_copy(
                table_spm.at[idx_all.at[pl.ds(i * C, C)]], rows[b], gsem[b])

        def store_desc(i, b):
            return pltpu.make_async_copy(
                rows[b], out_hbm.at[pl.ds(base + i * C, C)], ssem[b])

        for b in range(NBUF - 1):
            gather_desc(b, b).start()

        for b in range(NBUF):
            gather_desc(b, b).wait()
            store_desc(b, b).start()
            if b > 0:
                store_desc(b - 1, b - 1).wait()
            gather_desc(b + NBUF - 1, (b - 1) % NBUF).start()

        def body(g, carry):
            for b in range(NBUF):
                i = NBUF * g + b
                gather_desc(i, b).wait()
                store_desc(i, b).start()
                store_desc(i - 1, (b - 1) % NBUF).wait()
                gather_desc(i + NBUF - 1, (b - 1) % NBUF).start()
            return carry

        lax.fori_loop(1, NCHUNK // NBUF - 1, body, 0)

        for b in range(NBUF):
            i = NCHUNK - NBUF + b
            gather_desc(i, b).wait()
            store_desc(i, b).start()
            store_desc(i - 1, (b - 1) % NBUF).wait()
            if b == 0:
                gather_desc(i + NBUF - 1, (b - 1) % NBUF).start()
        store_desc(NCHUNK - 1, NBUF - 1).wait()

    return k(idx_flat, ne)


def kernel(node_ids, ne):
    idx_flat = node_ids.reshape(-1).astype(jnp.int32)
    out = _sc_gather(idx_flat, ne)
    return out.reshape(node_ids.shape + (D,))
